# baseline (speedup 1.0000x reference)
import jax
import jax.numpy as jnp
from jax.experimental import pallas as pl
from jax.experimental.pallas import tpu as pltpu

_BM = 512
_KC = 1024


def _mlp_kernel(coef_ref, x_hbm, w1_ref, b1_ref, w2_ref, b2_ref, out_ref,
                xv_ref, sem):
    i = pl.program_id(0)
    nsteps = pl.num_programs(0)
    slot = jax.lax.rem(i, 2)
    nslot = jax.lax.rem(i + 1, 2)

    @pl.when(i == 0)
    def _first():
        pltpu.make_async_copy(x_hbm.at[pl.ds(0, _BM)], xv_ref.at[0],
                              sem.at[0]).start()

    @pl.when(i + 1 < nsteps)
    def _prefetch():
        pltpu.make_async_copy(x_hbm.at[pl.ds((i + 1) * _BM, _BM)],
                              xv_ref.at[nslot], sem.at[nslot]).start()

    pltpu.make_async_copy(x_hbm.at[pl.ds(i * _BM, _BM)], xv_ref.at[slot],
                          sem.at[slot]).wait()

    T = x_hbm.shape[1]
    n = w1_ref.shape[1] // 2
    c = coef_ref[...].astype(jnp.bfloat16)
    h = None
    for c0 in range(0, T, _KC):
        xb = xv_ref[slot, :, c0:c0 + _KC]
        v0 = (xb & 1).astype(jnp.bfloat16)
        v1 = (xb >> 1).astype(jnp.bfloat16)
        p = v0 * v1
        g0 = c[0:1, 0:1] + c[0:1, 1:2] * v0 + c[0:1, 2:3] * v1 + c[0:1, 3:4] * p
        g1 = c[0:1, 4:5] + c[0:1, 5:6] * v0 + c[0:1, 6:7] * v1 + c[0:1, 7:8] * p
        d = jnp.dot(g0, w1_ref[c0:c0 + _KC, :n],
                    preferred_element_type=jnp.float32)
        d = d + jnp.dot(g1, w1_ref[c0:c0 + _KC, n:],
                        preferred_element_type=jnp.float32)
        h = d if h is None else h + d
    h = jnp.maximum(h + b1_ref[...], 0.0)
    out_ref[...] = jnp.dot(h, w2_ref[...],
                           preferred_element_type=jnp.float32) + b2_ref[...]


def kernel(x, table, W1, b1, W2, b2):
    B, T = x.shape
    d_hid = W1.shape[1]
    d_out = W2.shape[1]
    t = table
    coef = jnp.stack([
        t[0, 0], t[1, 0] - t[0, 0], t[2, 0] - t[0, 0],
        t[3, 0] - t[2, 0] - t[1, 0] + t[0, 0],
        t[0, 1], t[1, 1] - t[0, 1], t[2, 1] - t[0, 1],
        t[3, 1] - t[2, 1] - t[1, 1] + t[0, 1],
    ]).reshape(1, 8)
    w1r = W1.reshape(T, 2 * d_hid).astype(jnp.bfloat16)
    b1r = b1.reshape(1, d_hid)
    b2r = b2.reshape(1, d_out)
    return pl.pallas_call(
        _mlp_kernel,
        grid=(B // _BM,),
        in_specs=[
            pl.BlockSpec((1, 8), lambda i: (0, 0)),
            pl.BlockSpec(memory_space=pltpu.MemorySpace.HBM),
            pl.BlockSpec((T, 2 * d_hid), lambda i: (0, 0)),
            pl.BlockSpec((1, d_hid), lambda i: (0, 0)),
            pl.BlockSpec((d_hid, d_out), lambda i: (0, 0)),
            pl.BlockSpec((1, d_out), lambda i: (0, 0)),
        ],
        out_specs=pl.BlockSpec((_BM, d_out), lambda i: (i, 0)),
        out_shape=jax.ShapeDtypeStruct((B, d_out), jnp.float32),
        scratch_shapes=[
            pltpu.VMEM((2, _BM, T), jnp.int32),
            pltpu.SemaphoreType.DMA((2,)),
        ],
        compiler_params=pltpu.CompilerParams(
            dimension_semantics=("arbitrary",)),
    )(coef, x, w1r, b1r, W2, b2r)


# speedup vs baseline: 1.0082x; 1.0082x over previous
import jax
import jax.numpy as jnp
from jax.experimental import pallas as pl
from jax.experimental.pallas import tpu as pltpu

_BM = 512


_KC = 1024


def _mlp_kernel(coef_ref, x_ref, w1_ref, b1_ref, w2_ref, b2_ref, out_ref):
    T = x_ref.shape[1]
    n = w1_ref.shape[1] // 2
    c = coef_ref[...].astype(jnp.bfloat16)
    h = None
    for c0 in range(0, T, _KC):
        xb = x_ref[:, c0:c0 + _KC]
        v0 = (xb & 1).astype(jnp.bfloat16)
        v1 = (xb >> 1).astype(jnp.bfloat16)
        p = v0 * v1
        g0 = c[0:1, 0:1] + c[0:1, 1:2] * v0 + c[0:1, 2:3] * v1 + c[0:1, 3:4] * p
        g1 = c[0:1, 4:5] + c[0:1, 5:6] * v0 + c[0:1, 6:7] * v1 + c[0:1, 7:8] * p
        d = jnp.dot(g0, w1_ref[c0:c0 + _KC, :n],
                    preferred_element_type=jnp.float32)
        d = d + jnp.dot(g1, w1_ref[c0:c0 + _KC, n:],
                        preferred_element_type=jnp.float32)
        h = d if h is None else h + d
    h = jnp.maximum(h + b1_ref[...], 0.0)
    out_ref[...] = jnp.dot(h, w2_ref[...],
                           preferred_element_type=jnp.float32) + b2_ref[...]


def kernel(x, table, W1, b1, W2, b2):
    B, T = x.shape
    d_hid = W1.shape[1]
    d_out = W2.shape[1]
    t = table
    coef = jnp.stack([
        t[0, 0], t[1, 0] - t[0, 0], t[2, 0] - t[0, 0],
        t[3, 0] - t[2, 0] - t[1, 0] + t[0, 0],
        t[0, 1], t[1, 1] - t[0, 1], t[2, 1] - t[0, 1],
        t[3, 1] - t[2, 1] - t[1, 1] + t[0, 1],
    ]).reshape(1, 8)
    w1r = W1.reshape(T, 2 * d_hid).astype(jnp.bfloat16)
    b1r = b1.reshape(1, d_hid)
    b2r = b2.reshape(1, d_out)
    return pl.pallas_call(
        _mlp_kernel,
        grid=(B // _BM,),
        in_specs=[
            pl.BlockSpec((1, 8), lambda i: (0, 0)),
            pl.BlockSpec((_BM, T), lambda i: (i, 0)),
            pl.BlockSpec((T, 2 * d_hid), lambda i: (0, 0)),
            pl.BlockSpec((1, d_hid), lambda i: (0, 0)),
            pl.BlockSpec((d_hid, d_out), lambda i: (0, 0)),
            pl.BlockSpec((1, d_out), lambda i: (0, 0)),
        ],
        out_specs=pl.BlockSpec((_BM, d_out), lambda i: (i, 0)),
        out_shape=jax.ShapeDtypeStruct((B, d_out), jnp.float32),
        compiler_params=pltpu.CompilerParams(
            dimension_semantics=("arbitrary",)),
    )(coef, x, w1r, b1r, W2, b2r)
